# Initial kernel scaffold; baseline (speedup 1.0000x reference)
import jax
import jax.numpy as jnp
from jax.experimental import pallas as pl


def kernel(x, batch, B, Pi):
    raise NotImplementedError("write your pallas kernel here")


# speedup vs baseline: 28.9964x; 28.9964x over previous
import functools

import jax
import jax.numpy as jnp
from jax import lax
from jax.experimental import pallas as pl
from jax.experimental.pallas import tpu as pltpu
from jax.experimental.pallas import tpu_sc as plsc

N = 800000
C = 8
M = 128
NGEN = 4
NSEG = 512

NC = 2
NS = 16
NW = NC * NS
PER_TILE = N // NW
BK = 5000
NBLK = PER_TILE // BK
FULLV = BK // 16
TAILR = BK - FULLV * 16
ACCL = NSEG * NGEN


def _tables_body(b_ref, pi_ref, lik_ref, hmax_ref, hidx_ref):
    bt = b_ref[...]
    bm = jnp.max(bt, axis=2, keepdims=True)
    be = jnp.exp(bt - bm)
    sm_b = be / jnp.sum(be, axis=2, keepdims=True)

    pi = pi_ref[...]
    pm = jnp.max(pi, axis=1, keepdims=True)
    pe = jnp.exp(pi - pm)
    sm_pi = pe / jnp.sum(pe, axis=1, keepdims=True)

    num = sm_pi[:, :, None] * sm_b
    post = num / jnp.sum(num, axis=1, keepdims=True)
    lik_ref[...] = jnp.sum(post * jnp.log(num), axis=1)
    hmax = jnp.max(post, axis=1)
    hmax_ref[...] = hmax
    ci = lax.broadcasted_iota(jnp.int32, (NGEN, C, M), 1)
    hidx_ref[...] = jnp.min(
        jnp.where(post == hmax[:, None, :], ci, C), axis=1
    ).astype(jnp.int32)


_tables_call = pl.pallas_call(
    _tables_body,
    out_shape=[
        jax.ShapeDtypeStruct((NGEN, M), jnp.float32),
        jax.ShapeDtypeStruct((NGEN, M), jnp.float32),
        jax.ShapeDtypeStruct((NGEN, M), jnp.int32),
    ],
)


def _combine_body(p_ref, out_ref):
    out_ref[...] = jnp.sum(p_ref[...], axis=0, keepdims=True)


_combine_call = pl.pallas_call(
    _combine_body,
    out_shape=jax.ShapeDtypeStruct((1, ACCL), jnp.float32),
)


@functools.cache
def _build_sc_main():
    mesh = plsc.VectorSubcoreMesh(
        core_axis_name="c", subcore_axis_name="s", num_cores=NC, num_subcores=NS
    )

    @functools.partial(
        pl.kernel,
        mesh=mesh,
        compiler_params=pltpu.CompilerParams(needs_layout_passes=False),
        out_type=[
            jax.ShapeDtypeStruct((N * NGEN,), jnp.float32),
            jax.ShapeDtypeStruct((N * NGEN,), jnp.int32),
            jax.ShapeDtypeStruct((NW, ACCL), jnp.float32),
        ],
        scratch_types=[
            pltpu.VMEM((M * NGEN,), jnp.float32),
            pltpu.VMEM((M * NGEN,), jnp.float32),
            pltpu.VMEM((M * NGEN,), jnp.int32),
            pltpu.VMEM((BK + 16,), jnp.int32),
            pltpu.VMEM((BK + 16,), jnp.int32),
            pltpu.VMEM((BK * NGEN,), jnp.float32),
            pltpu.VMEM((BK * NGEN,), jnp.int32),
            pltpu.VMEM((ACCL,), jnp.float32),
            pltpu.SemaphoreType.DMA,
        ],
    )
    def _sc_main(x_hbm, batch_hbm, tlik_hbm, thmax_hbm, thidx_hbm,
                 hmax_out, hidx_out, part_out,
                 tlik_v, thmax_v, thidx_v, x_v, b_v, hm_v, hi_v, acc_v, sem):
        wid = lax.axis_index("s") * NC + lax.axis_index("c")
        base = wid * PER_TILE

        pltpu.sync_copy(tlik_hbm, tlik_v)
        pltpu.sync_copy(thmax_hbm, thmax_v)
        pltpu.sync_copy(thidx_hbm, thidx_v)

        zeros16 = jnp.zeros((16,), jnp.float32)

        def _zero(i, carry):
            acc_v[pl.ds(i * 16, 16)] = zeros16
            return carry

        lax.fori_loop(0, ACCL // 16, _zero, 0)

        iota = lax.iota(jnp.int32, 16)
        iota4 = iota * 4
        l15 = iota == 15
        lt15 = iota < 15

        def _vec(s, tail):
            xv = x_v[pl.ds(s, 16)]
            bv = b_v[pl.ds(s, 16)]
            bn = b_v[pl.ds(s + 1, 16)]
            if tail:
                valid = iota < TAILR
                xv = jnp.where(valid, xv, 0)
                bv = jnp.where(valid, bv, 0)
                neq = (bv != bn) & (iota < TAILR - 1)
                ends = neq | (iota == TAILR - 1)
            else:
                valid = None
                neq = (bv != bn) & lt15
                ends = neq | l15
            bn = jnp.where(neq, bn, 0)
            x4 = xv * 4
            b4 = bv * 4
            bn4 = bn * 4
            obase = iota4 + s * 4
            if tail:
                obase = jnp.where(valid, obase, 0)
            for g in range(NGEN):
                idxg = x4 + g if g else x4
                likv = plsc.load_gather(tlik_v, [idxg], mask=valid)
                if tail:
                    likv = jnp.where(valid, likv, 0.0)
                cs = plsc.cumsum(likv)
                bidx = b4 + g if g else b4
                bnidx = bn4 + g if g else bn4
                plsc.addupdate_scatter(acc_v, [bidx], cs, mask=ends)
                plsc.addupdate_scatter(acc_v, [bnidx], -cs, mask=neq)
                hv = plsc.load_gather(thmax_v, [idxg], mask=valid)
                iv = plsc.load_gather(thidx_v, [idxg], mask=valid)
                oidx = obase + g if g else obase
                plsc.store_scatter(hm_v, [oidx], hv, mask=valid)
                plsc.store_scatter(hi_v, [oidx], iv, mask=valid)

        def _vec_body(i, carry):
            _vec(i * 16, tail=False)
            return carry

        for blk in range(NBLK):
            off = pl.multiple_of(base + blk * BK, 8)
            pltpu.sync_copy(x_hbm.at[pl.ds(off, BK)], x_v.at[pl.ds(0, BK)])
            pltpu.sync_copy(batch_hbm.at[pl.ds(off, BK)], b_v.at[pl.ds(0, BK)])
            lax.fori_loop(0, FULLV, _vec_body, 0)
            _vec(FULLV * 16, tail=True)
            off4 = pl.multiple_of(off * 4, 8)
            pltpu.sync_copy(hm_v, hmax_out.at[pl.ds(off4, BK * 4)])
            pltpu.sync_copy(hi_v, hidx_out.at[pl.ds(off4, BK * 4)])

        pltpu.sync_copy(acc_v, part_out.at[wid])

    return _sc_main


def kernel(x, batch, B, Pi):
    bt = jnp.transpose(B, (2, 0, 1))
    pit = jnp.transpose(Pi, (1, 0))
    lik_t, hmax_t, hidx_t = _tables_call(bt, pit)
    tlik = jnp.reshape(jnp.transpose(lik_t), (M * NGEN,))
    thmax = jnp.reshape(jnp.transpose(hmax_t), (M * NGEN,))
    thidx = jnp.reshape(jnp.transpose(hidx_t), (M * NGEN,))
    hm_flat, hi_flat, parts = _build_sc_main()(x, batch, tlik, thmax, thidx)
    lik_sum = _combine_call(parts)
    likelihood = jnp.reshape(lik_sum, (NSEG, NGEN))
    h_max_vals = jnp.reshape(hm_flat, (N, 1, NGEN))
    h_max_idx = jnp.reshape(hi_flat, (N, NGEN))
    return (likelihood, h_max_vals, h_max_idx)


# speedup vs baseline: 178.1476x; 6.1438x over previous
import functools

import jax
import jax.numpy as jnp
from jax import lax
from jax.experimental import pallas as pl
from jax.experimental.pallas import tpu as pltpu
from jax.experimental.pallas import tpu_sc as plsc

N = 800000
C = 8
M = 128
NGEN = 4
NSEG = 512

NC = 2
NS = 16
NW = NC * NS
GRP = 128
NGRP = N // GRP
GPW = NGRP // NW
XTRA = NGRP - GPW * NW
GPB = 39
NBLK = GPW // GPB
BK = GPB * GRP
ACCL = NSEG * NGEN


def _tables_body(b_ref, pi_ref, lik_ref, hmax_ref, hidx_ref):
    bt = b_ref[...]
    bm = jnp.max(bt, axis=2, keepdims=True)
    be = jnp.exp(bt - bm)
    sm_b = be / jnp.sum(be, axis=2, keepdims=True)

    pi = pi_ref[...]
    pm = jnp.max(pi, axis=1, keepdims=True)
    pe = jnp.exp(pi - pm)
    sm_pi = pe / jnp.sum(pe, axis=1, keepdims=True)

    num = sm_pi[:, :, None] * sm_b
    post = num / jnp.sum(num, axis=1, keepdims=True)
    lik_ref[...] = jnp.sum(post * jnp.log(num), axis=1)
    hmax = jnp.max(post, axis=1)
    hmax_ref[...] = hmax
    ci = lax.broadcasted_iota(jnp.int32, (NGEN, C, M), 1)
    hidx_ref[...] = jnp.min(
        jnp.where(post == hmax[:, None, :], ci, C), axis=1
    ).astype(jnp.int32)


_tables_call = pl.pallas_call(
    _tables_body,
    out_shape=[
        jax.ShapeDtypeStruct((NGEN, M), jnp.float32),
        jax.ShapeDtypeStruct((NGEN, M), jnp.float32),
        jax.ShapeDtypeStruct((NGEN, M), jnp.int32),
    ],
)


def _combine_body(p_ref, out_ref):
    out_ref[...] = jnp.sum(p_ref[...], axis=0, keepdims=True)


_combine_call = pl.pallas_call(
    _combine_body,
    out_shape=jax.ShapeDtypeStruct((1, ACCL), jnp.float32),
)


@functools.cache
def _build_sc_main():
    mesh = plsc.VectorSubcoreMesh(
        core_axis_name="c", subcore_axis_name="s", num_cores=NC, num_subcores=NS
    )

    @functools.partial(
        pl.kernel,
        mesh=mesh,
        compiler_params=pltpu.CompilerParams(needs_layout_passes=False),
        out_type=[
            jax.ShapeDtypeStruct((N * NGEN,), jnp.float32),
            jax.ShapeDtypeStruct((N * NGEN,), jnp.int32),
            jax.ShapeDtypeStruct((NW, ACCL), jnp.float32),
        ],
        scratch_types=[
            pltpu.VMEM((M * NGEN,), jnp.float32),
            pltpu.VMEM((M * NGEN,), jnp.float32),
            pltpu.VMEM((M * NGEN,), jnp.int32),
            pltpu.VMEM((BK + 16,), jnp.int32),
            pltpu.VMEM((BK + 16,), jnp.int32),
            pltpu.VMEM((BK * NGEN,), jnp.float32),
            pltpu.VMEM((BK * NGEN,), jnp.int32),
            pltpu.VMEM((ACCL,), jnp.float32),
            pltpu.SemaphoreType.DMA,
        ],
    )
    def _sc_main(x_hbm, batch_hbm, tlik_hbm, thmax_hbm, thidx_hbm,
                 hmax_out, hidx_out, part_out,
                 tlik_v, thmax_v, thidx_v, x_v, b_v, hm_v, hi_v, acc_v, sem):
        wid = lax.axis_index("s") * NC + lax.axis_index("c")
        base = (wid * GPW + jnp.minimum(wid, XTRA)) * GRP

        pltpu.sync_copy(tlik_hbm, tlik_v)
        pltpu.sync_copy(thmax_hbm, thmax_v)
        pltpu.sync_copy(thidx_hbm, thidx_v)

        zeros16 = jnp.zeros((16,), jnp.float32)

        def _zero(i, carry):
            acc_v[pl.ds(i * 16, 16)] = zeros16
            return carry

        lax.fori_loop(0, ACCL // 16, _zero, 0)

        iota = lax.iota(jnp.int32, 16)
        l15 = iota == 15
        lt15 = iota < 15

        def _vec(s):
            xv = x_v[pl.ds(s, 16)]
            bv = b_v[pl.ds(s, 16)]
            bn = b_v[pl.ds(s + 1, 16)]
            neq = (bv != bn) & lt15
            ends = neq | l15
            bn = jnp.where(neq, bn, 0)
            x4 = xv * 4
            b4 = bv * 4
            bn4 = bn * 4
            og = ((s >> 7) << 9) + (s & 127)
            for g in range(NGEN):
                idxg = x4 + g if g else x4
                likv = plsc.load_gather(tlik_v, [idxg])
                cs = plsc.cumsum(likv)
                bidx = b4 + g if g else b4
                bnidx = bn4 + g if g else bn4
                plsc.addupdate_scatter(acc_v, [bidx], cs, mask=ends)
                plsc.addupdate_scatter(acc_v, [bnidx], -cs, mask=neq)
                hv = plsc.load_gather(thmax_v, [idxg])
                iv = plsc.load_gather(thidx_v, [idxg])
                hm_v[pl.ds(og + g * GRP, 16)] = hv
                hi_v[pl.ds(og + g * GRP, 16)] = iv

        def _vec_body(i, carry):
            _vec(i * 16)
            return carry

        def _block(off, gcount):
            bk = gcount * GRP
            pltpu.sync_copy(x_hbm.at[pl.ds(off, bk)], x_v.at[pl.ds(0, bk)])
            pltpu.sync_copy(batch_hbm.at[pl.ds(off, bk)], b_v.at[pl.ds(0, bk)])
            lax.fori_loop(0, bk // 16, _vec_body, 0)
            off4 = pl.multiple_of(off * NGEN, 8)
            pltpu.sync_copy(hm_v.at[pl.ds(0, bk * NGEN)],
                            hmax_out.at[pl.ds(off4, bk * NGEN)])
            pltpu.sync_copy(hi_v.at[pl.ds(0, bk * NGEN)],
                            hidx_out.at[pl.ds(off4, bk * NGEN)])

        for blk in range(NBLK):
            _block(pl.multiple_of(base + blk * BK, 8), GPB)

        @pl.when(wid < XTRA)
        def _():
            _block(pl.multiple_of(base + GPW * GRP, 8), 1)

        pltpu.sync_copy(acc_v, part_out.at[wid])

    return _sc_main


def kernel(x, batch, B, Pi):
    bt = jnp.transpose(B, (2, 0, 1))
    pit = jnp.transpose(Pi, (1, 0))
    lik_t, hmax_t, hidx_t = _tables_call(bt, pit)
    tlik = jnp.reshape(jnp.transpose(lik_t), (M * NGEN,))
    thmax = jnp.reshape(jnp.transpose(hmax_t), (M * NGEN,))
    thidx = jnp.reshape(jnp.transpose(hidx_t), (M * NGEN,))
    hm_flat, hi_flat, parts = _build_sc_main()(x, batch, tlik, thmax, thidx)
    lik_sum = _combine_call(parts)
    likelihood = jnp.reshape(lik_sum, (NSEG, NGEN))
    h_max_vals = jnp.reshape(
        jnp.transpose(jnp.reshape(hm_flat, (NGRP, NGEN, GRP)), (0, 2, 1)),
        (N, 1, NGEN))
    h_max_idx = jnp.reshape(
        jnp.transpose(jnp.reshape(hi_flat, (NGRP, NGEN, GRP)), (0, 2, 1)),
        (N, NGEN))
    return (likelihood, h_max_vals, h_max_idx)


# speedup vs baseline: 288.7389x; 1.6208x over previous
import functools

import jax
import jax.numpy as jnp
from jax import lax
from jax.experimental import pallas as pl
from jax.experimental.pallas import tpu as pltpu
from jax.experimental.pallas import tpu_sc as plsc

N = 800000
C = 8
M = 128
NGEN = 4
NSEG = 512

NC = 2
NS = 16
NW = NC * NS
GRP = 128
NGRP = N // GRP
GPW = NGRP // NW
XTRA = NGRP - GPW * NW
GPB = 39
NBLK = GPW // GPB
BK = GPB * GRP
ACCL = NSEG * NGEN


def _tables_body(b_ref, pi_ref, lik_ref, pack_ref):
    bt = b_ref[...]
    bm = jnp.max(bt, axis=2, keepdims=True)
    be = jnp.exp(bt - bm)
    sm_b = be / jnp.sum(be, axis=2, keepdims=True)

    pi = pi_ref[...]
    pm = jnp.max(pi, axis=1, keepdims=True)
    pe = jnp.exp(pi - pm)
    sm_pi = pe / jnp.sum(pe, axis=1, keepdims=True)

    num = sm_pi[:, :, None] * sm_b
    post = num / jnp.sum(num, axis=1, keepdims=True)
    lik_ref[...] = jnp.sum(post * jnp.log(num), axis=1)
    hmax = jnp.max(post, axis=1)
    ci = lax.broadcasted_iota(jnp.int32, (NGEN, C, M), 1)
    hidx = jnp.min(jnp.where(post == hmax[:, None, :], ci, C), axis=1)
    hbits = lax.bitcast_convert_type(hmax, jnp.int32)
    pack_ref[...] = lax.bitcast_convert_type(
        (hbits & -8) | hidx.astype(jnp.int32), jnp.float32)


_tables_call = pl.pallas_call(
    _tables_body,
    out_shape=[
        jax.ShapeDtypeStruct((NGEN, M), jnp.float32),
        jax.ShapeDtypeStruct((NGEN, M), jnp.float32),
    ],
)


def _combine_body(p_ref, out_ref):
    out_ref[...] = jnp.sum(p_ref[...], axis=0, keepdims=True)


_combine_call = pl.pallas_call(
    _combine_body,
    out_shape=jax.ShapeDtypeStruct((1, ACCL), jnp.float32),
)


@functools.cache
def _build_sc_main():
    mesh = plsc.VectorSubcoreMesh(
        core_axis_name="c", subcore_axis_name="s", num_cores=NC, num_subcores=NS
    )

    @functools.partial(
        pl.kernel,
        mesh=mesh,
        compiler_params=pltpu.CompilerParams(needs_layout_passes=False),
        out_type=[
            jax.ShapeDtypeStruct((N * NGEN,), jnp.float32),
            jax.ShapeDtypeStruct((N * NGEN,), jnp.int32),
            jax.ShapeDtypeStruct((NW, ACCL), jnp.float32),
        ],
        scratch_types=[
            pltpu.VMEM((M * NGEN,), jnp.float32),
            pltpu.VMEM((M * NGEN,), jnp.float32),
            pltpu.VMEM((BK + 16,), jnp.int32),
            pltpu.VMEM((BK + 16,), jnp.int32),
            pltpu.VMEM((BK * NGEN,), jnp.float32),
            pltpu.VMEM((BK * NGEN,), jnp.int32),
            pltpu.VMEM((ACCL,), jnp.float32),
            pltpu.SemaphoreType.DMA,
        ],
    )
    def _sc_main(x_hbm, batch_hbm, tlik_hbm, tpack_hbm,
                 hmax_out, hidx_out, part_out,
                 tlik_v, tpack_v, x_v, b_v, hm_v, hi_v, acc_v, sem):
        wid = lax.axis_index("s") * NC + lax.axis_index("c")
        base = (wid * GPW + jnp.minimum(wid, XTRA)) * GRP

        pltpu.sync_copy(tlik_hbm, tlik_v)
        pltpu.sync_copy(tpack_hbm, tpack_v)

        zeros16 = jnp.zeros((16,), jnp.float32)

        def _zero(i, carry):
            acc_v[pl.ds(i * 16, 16)] = zeros16
            return carry

        lax.fori_loop(0, ACCL // 16, _zero, 0)

        iota = lax.iota(jnp.int32, 16)
        l15 = iota == 15
        lt15 = iota < 15

        def _vec_body(i, rs):
            s = i * 16
            xv = x_v[pl.ds(s, 16)]
            bv = b_v[pl.ds(s, 16)]
            bn = b_v[pl.ds(s + 1, 16)]
            neq_full = bv != bn
            neq = neq_full & lt15
            x4 = xv * 4
            og = ((s >> 7) << 9) + (s & 127)
            likvs = []
            for g in range(NGEN):
                idxg = x4 + g if g else x4
                likvs.append(plsc.load_gather(tlik_v, [idxg]))
                pv = plsc.load_gather(tpack_v, [idxg])
                hm_v[pl.ds(og + g * GRP, 16)] = pv
                hi_v[pl.ds(og + g * GRP, 16)] = plsc.bitcast(pv, jnp.int32) & 7
            nb = plsc.all_reduce_population_count(neq_full)
            anyb = jnp.max(nb)

            def _fast(rs):
                return tuple(r + l for r, l in zip(rs, likvs))

            def _slow(rs):
                b4 = bv * 4
                bn4 = jnp.where(neq, bn, 0) * 4
                ends = neq | l15
                for g in range(NGEN):
                    cs = plsc.cumsum(likvs[g]) + jnp.sum(rs[g])
                    bidx = b4 + g if g else b4
                    bnidx = bn4 + g if g else bn4
                    plsc.addupdate_scatter(acc_v, [bidx], cs, mask=ends)
                    plsc.addupdate_scatter(acc_v, [bnidx], -cs, mask=neq)
                return tuple(zeros16 for _ in range(NGEN))

            return lax.cond(anyb > 0, _slow, _fast, rs)

        def _block(off, gcount):
            bk = gcount * GRP
            pltpu.sync_copy(x_hbm.at[pl.ds(off, bk)], x_v.at[pl.ds(0, bk)])
            pltpu.sync_copy(batch_hbm.at[pl.ds(off, bk)], b_v.at[pl.ds(0, bk)])
            rs0 = tuple(zeros16 for _ in range(NGEN))
            rs = lax.fori_loop(0, bk // 16, _vec_body, rs0)
            bl4 = b_v[pl.ds(bk - 16, 16)] * 4
            for g in range(NGEN):
                tot = jnp.full((16,), jnp.sum(rs[g]), jnp.float32)
                bidx = bl4 + g if g else bl4
                plsc.addupdate_scatter(acc_v, [bidx], tot, mask=l15)
            off4 = pl.multiple_of(off * NGEN, 8)
            pltpu.sync_copy(hm_v.at[pl.ds(0, bk * NGEN)],
                            hmax_out.at[pl.ds(off4, bk * NGEN)])
            pltpu.sync_copy(hi_v.at[pl.ds(0, bk * NGEN)],
                            hidx_out.at[pl.ds(off4, bk * NGEN)])

        for blk in range(NBLK):
            _block(pl.multiple_of(base + blk * BK, 8), GPB)

        @pl.when(wid < XTRA)
        def _():
            _block(pl.multiple_of(base + GPW * GRP, 8), 1)

        pltpu.sync_copy(acc_v, part_out.at[wid])

    return _sc_main


def kernel(x, batch, B, Pi):
    bt = jnp.transpose(B, (2, 0, 1))
    pit = jnp.transpose(Pi, (1, 0))
    lik_t, pack_t = _tables_call(bt, pit)
    tlik = jnp.reshape(jnp.transpose(lik_t), (M * NGEN,))
    tpack = jnp.reshape(jnp.transpose(pack_t), (M * NGEN,))
    hm_flat, hi_flat, parts = _build_sc_main()(x, batch, tlik, tpack)
    lik_sum = _combine_call(parts)
    likelihood = jnp.reshape(lik_sum, (NSEG, NGEN))
    h_max_vals = jnp.reshape(
        jnp.transpose(jnp.reshape(hm_flat, (NGRP, NGEN, GRP)), (0, 2, 1)),
        (N, 1, NGEN))
    h_max_idx = jnp.reshape(
        jnp.transpose(jnp.reshape(hi_flat, (NGRP, NGEN, GRP)), (0, 2, 1)),
        (N, NGEN))
    return (likelihood, h_max_vals, h_max_idx)


# speedup vs baseline: 304.8088x; 1.0557x over previous
import functools

import jax
import jax.numpy as jnp
from jax import lax
from jax.experimental import pallas as pl
from jax.experimental.pallas import tpu as pltpu
from jax.experimental.pallas import tpu_sc as plsc

N = 800000
C = 8
M = 128
NGEN = 4
NSEG = 512

NC = 2
NS = 16
NW = NC * NS
GRP = 128
NGRP = N // GRP
GPW = NGRP // NW
XTRA = NGRP - GPW * NW
GPB = 39
NBLK = GPW // GPB
BK = GPB * GRP
ACCL = NSEG * NGEN


def _tables_body(b_ref, pi_ref, lik_ref, pack_ref):
    bt = b_ref[...]
    bm = jnp.max(bt, axis=2, keepdims=True)
    be = jnp.exp(bt - bm)
    sm_b = be / jnp.sum(be, axis=2, keepdims=True)

    pi = pi_ref[...]
    pm = jnp.max(pi, axis=1, keepdims=True)
    pe = jnp.exp(pi - pm)
    sm_pi = pe / jnp.sum(pe, axis=1, keepdims=True)

    num = sm_pi[:, :, None] * sm_b
    post = num / jnp.sum(num, axis=1, keepdims=True)
    lik_ref[...] = jnp.sum(post * jnp.log(num), axis=1)
    hmax = jnp.max(post, axis=1)
    ci = lax.broadcasted_iota(jnp.int32, (NGEN, C, M), 1)
    hidx = jnp.min(jnp.where(post == hmax[:, None, :], ci, C), axis=1)
    hbits = lax.bitcast_convert_type(hmax, jnp.int32)
    pack_ref[...] = lax.bitcast_convert_type(
        (hbits & -8) | hidx.astype(jnp.int32), jnp.float32)


_tables_call = pl.pallas_call(
    _tables_body,
    out_shape=[
        jax.ShapeDtypeStruct((NGEN, M), jnp.float32),
        jax.ShapeDtypeStruct((NGEN, M), jnp.float32),
    ],
)


def _combine_body(p_ref, out_ref):
    out_ref[...] = jnp.sum(p_ref[...], axis=0, keepdims=True)


_combine_call = pl.pallas_call(
    _combine_body,
    out_shape=jax.ShapeDtypeStruct((1, ACCL), jnp.float32),
)


@functools.cache
def _build_sc_main():
    mesh = plsc.VectorSubcoreMesh(
        core_axis_name="c", subcore_axis_name="s", num_cores=NC, num_subcores=NS
    )

    @functools.partial(
        pl.kernel,
        mesh=mesh,
        compiler_params=pltpu.CompilerParams(needs_layout_passes=False),
        out_type=[
            jax.ShapeDtypeStruct((N * NGEN,), jnp.float32),
            jax.ShapeDtypeStruct((N * NGEN,), jnp.int32),
            jax.ShapeDtypeStruct((NW, ACCL), jnp.float32),
        ],
        scratch_types=[
            pltpu.VMEM((M * NGEN,), jnp.float32),
            pltpu.VMEM((M * NGEN,), jnp.float32),
            pltpu.VMEM((BK + 16,), jnp.int32),
            pltpu.VMEM((BK + 16,), jnp.int32),
            pltpu.VMEM((BK * NGEN,), jnp.float32),
            pltpu.VMEM((BK * NGEN,), jnp.int32),
            pltpu.VMEM((ACCL,), jnp.float32),
            pltpu.SemaphoreType.DMA,
        ],
    )
    def _sc_main(x_hbm, batch_hbm, tlik_hbm, tpack_hbm,
                 hmax_out, hidx_out, part_out,
                 tlik_v, tpack_v, x_v, b_v, hm_v, hi_v, acc_v, sem):
        wid = lax.axis_index("s") * NC + lax.axis_index("c")
        base = (wid * GPW + jnp.minimum(wid, XTRA)) * GRP

        pltpu.sync_copy(tlik_hbm, tlik_v)
        pltpu.sync_copy(tpack_hbm, tpack_v)

        zeros16 = jnp.zeros((16,), jnp.float32)

        def _zero(i, carry):
            acc_v[pl.ds(i * 16, 16)] = zeros16
            return carry

        lax.fori_loop(0, ACCL // 16, _zero, 0)

        iota = lax.iota(jnp.int32, 16)
        l15 = iota == 15
        lt15 = iota < 15

        def _vec_body(i, rs):
            s = i * 16
            xv = x_v[pl.ds(s, 16)]
            bv = b_v[pl.ds(s, 16)]
            bn = b_v[pl.ds(s + 1, 16)]
            neq_full = bv != bn
            neq = neq_full & lt15
            x4 = xv * 4
            og = ((s >> 7) << 9) + (s & 127)
            likvs = []
            for g in range(NGEN):
                idxg = x4 + g if g else x4
                likvs.append(plsc.load_gather(tlik_v, [idxg]))
                pv = plsc.load_gather(tpack_v, [idxg])
                hm_v[pl.ds(og + g * GRP, 16)] = pv
                hi_v[pl.ds(og + g * GRP, 16)] = plsc.bitcast(pv, jnp.int32) & 7
            anyb = bv[0] != bn[15]

            def _fast(rs):
                return tuple(r + l for r, l in zip(rs, likvs))

            def _slow(rs):
                b4 = bv * 4
                bn4 = jnp.where(neq, bn, 0) * 4
                ends = neq | l15
                for g in range(NGEN):
                    cs = plsc.cumsum(likvs[g]) + jnp.sum(rs[g])
                    bidx = b4 + g if g else b4
                    bnidx = bn4 + g if g else bn4
                    plsc.addupdate_scatter(acc_v, [bidx], cs, mask=ends)
                    plsc.addupdate_scatter(acc_v, [bnidx], -cs, mask=neq)
                return tuple(zeros16 for _ in range(NGEN))

            return lax.cond(anyb, _slow, _fast, rs)

        def _block(off, gcount):
            bk = gcount * GRP
            pltpu.sync_copy(x_hbm.at[pl.ds(off, bk)], x_v.at[pl.ds(0, bk)])
            pltpu.sync_copy(batch_hbm.at[pl.ds(off, bk)], b_v.at[pl.ds(0, bk)])
            b_v[pl.ds(bk, 16)] = jnp.full((16,), jnp.int32(2**31 - 1))
            rs0 = tuple(zeros16 for _ in range(NGEN))
            rs = lax.fori_loop(0, bk // 16, _vec_body, rs0)
            bl4 = b_v[pl.ds(bk - 16, 16)] * 4
            for g in range(NGEN):
                tot = jnp.full((16,), jnp.sum(rs[g]), jnp.float32)
                bidx = bl4 + g if g else bl4
                plsc.addupdate_scatter(acc_v, [bidx], tot, mask=l15)
            off4 = pl.multiple_of(off * NGEN, 8)
            pltpu.sync_copy(hm_v.at[pl.ds(0, bk * NGEN)],
                            hmax_out.at[pl.ds(off4, bk * NGEN)])
            pltpu.sync_copy(hi_v.at[pl.ds(0, bk * NGEN)],
                            hidx_out.at[pl.ds(off4, bk * NGEN)])

        for blk in range(NBLK):
            _block(pl.multiple_of(base + blk * BK, 8), GPB)

        @pl.when(wid < XTRA)
        def _():
            _block(pl.multiple_of(base + GPW * GRP, 8), 1)

        pltpu.sync_copy(acc_v, part_out.at[wid])

    return _sc_main


def kernel(x, batch, B, Pi):
    bt = jnp.transpose(B, (2, 0, 1))
    pit = jnp.transpose(Pi, (1, 0))
    lik_t, pack_t = _tables_call(bt, pit)
    tlik = jnp.reshape(jnp.transpose(lik_t), (M * NGEN,))
    tpack = jnp.reshape(jnp.transpose(pack_t), (M * NGEN,))
    hm_flat, hi_flat, parts = _build_sc_main()(x, batch, tlik, tpack)
    lik_sum = _combine_call(parts)
    likelihood = jnp.reshape(lik_sum, (NSEG, NGEN))
    h_max_vals = jnp.reshape(
        jnp.transpose(jnp.reshape(hm_flat, (NGRP, NGEN, GRP)), (0, 2, 1)),
        (N, 1, NGEN))
    h_max_idx = jnp.reshape(
        jnp.transpose(jnp.reshape(hi_flat, (NGRP, NGEN, GRP)), (0, 2, 1)),
        (N, NGEN))
    return (likelihood, h_max_vals, h_max_idx)
